# baseline (speedup 1.0000x reference)
import functools

import jax
import jax.numpy as jnp
from jax import lax
from jax.experimental import pallas as pl
from jax.experimental.pallas import tpu as pltpu
from jax.experimental.pallas import tpu_sc as plsc

N_NODES = 100000
F = 128
NUM_TILES = 16
EPT = N_NODES // NUM_TILES
SCAT = 125
NBATCH = EPT // SCAT
CH = 625
NCHUNK = EPT // CH
BPC = CH // SCAT
SW = 32
NSLICE = F // SW
ACC_ROWS = N_NODES + 8


def _sc_segment_sums(xbf, xcat, parent3d):
  mesh = plsc.VectorSubcoreMesh(core_axis_name="c", subcore_axis_name="s")

  @functools.partial(
      pl.kernel,
      out_type=[
          jax.ShapeDtypeStruct((N_NODES, F), jnp.bfloat16),
          jax.ShapeDtypeStruct((N_NODES, F), jnp.bfloat16),
      ],
      mesh=mesh,
      compiler_params=pltpu.CompilerParams(use_tc_tiling_on_sc=False,
                                           needs_layout_passes=False),
      scratch_types=[
          pltpu.VMEM_SHARED((ACC_ROWS, SW), jnp.bfloat16),
          pltpu.VMEM((CH, SW), jnp.bfloat16),
          pltpu.VMEM((CH, SW), jnp.bfloat16),
          pltpu.VMEM((NBATCH, SCAT), jnp.int32),
          pltpu.SemaphoreType.DMA,
          pltpu.SemaphoreType.DMA,
          pltpu.SemaphoreType.DMA,
      ],
  )
  def k(x_hbm, xcat_hbm, par_hbm, s_out, r_out, acc, buf0, buf1, idxbuf,
        sem_a, sem_b, sem_s):
    c = lax.axis_index("c")
    s = lax.axis_index("s")
    lo = s * EPT

    pltpu.sync_copy(par_hbm.at[s], idxbuf)

    def init_my_slab(fs):
      @pl.when(c == 0)
      def _():
        pltpu.sync_copy(x_hbm.at[pl.ds(lo, EPT), pl.ds(fs * SW, SW)],
                        acc.at[pl.ds(lo, EPT)])

      @pl.when(c == 1)
      def _():
        pltpu.sync_copy(xcat_hbm.at[pl.ds(lo, EPT), pl.ds(F + fs * SW, SW)],
                        acc.at[pl.ds(lo, EPT)])

    init_my_slab(0)

    for fs in range(NSLICE):
      plsc.subcore_barrier()

      src = [x_hbm, xcat_hbm]
      col = pl.ds(fs * SW, SW)

      @pl.loop(0, NCHUNK // 2)
      def _(p):
        a = 2 * p
        b = 2 * p + 1
        for cc in range(2):
          @pl.when(c == cc)
          def _():
            pltpu.async_copy(
                src[cc].at[pl.ds(lo + a * CH, CH), col], buf0, sem_a)
            pltpu.async_copy(
                src[cc].at[pl.ds(lo + b * CH, CH), col], buf1, sem_b)

        pltpu.make_async_copy(
            x_hbm.at[pl.ds(lo + a * CH, CH), col], buf0, sem_a).wait()
        sc = [pltpu.async_copy(buf0.at[pl.ds(j * SCAT, SCAT)],
                               acc.at[idxbuf.at[a * BPC + j]], sem_s,
                               add=True)
              for j in range(BPC)]
        pltpu.make_async_copy(
            x_hbm.at[pl.ds(lo + b * CH, CH), col], buf1, sem_b).wait()
        sc += [pltpu.async_copy(buf1.at[pl.ds(j * SCAT, SCAT)],
                                acc.at[idxbuf.at[b * BPC + j]], sem_s,
                                add=True)
               for j in range(BPC)]
        for d in sc:
          d.wait()

      plsc.subcore_barrier()

      @pl.when(c == 0)
      def _():
        pltpu.sync_copy(acc.at[pl.ds(lo, EPT)],
                        s_out.at[pl.ds(lo, EPT), pl.ds(fs * SW, SW)])

      @pl.when(c == 1)
      def _():
        pltpu.sync_copy(acc.at[pl.ds(lo, EPT)],
                        r_out.at[pl.ds(lo, EPT), pl.ds(fs * SW, SW)])

      if fs != NSLICE - 1:
        init_my_slab(fs + 1)

  return k(xbf, xcat, parent3d)


def _scale_body(x_ref, we_ref, w0_ref, xb_ref, o_ref):
  xb = x_ref[...]
  xb_ref[...] = xb.astype(jnp.bfloat16)
  o_ref[:, :F] = (xb * we_ref[0, 0, :][:, None]).astype(jnp.bfloat16)
  o_ref[:, F:] = (xb * w0_ref[0, 0, :][:, None]).astype(jnp.bfloat16)


def _tc_scale(x, w_edge, w_ident):
  blk = 2000
  grid = N_NODES // blk
  vec_spec = pl.BlockSpec((1, 1, blk), lambda i: (i, 0, 0))
  return pl.pallas_call(
      _scale_body,
      grid=(grid,),
      in_specs=[pl.BlockSpec((blk, F), lambda i: (i, 0)), vec_spec, vec_spec],
      out_specs=[pl.BlockSpec((blk, F), lambda i: (i, 0)),
                 pl.BlockSpec((blk, 2 * F), lambda i: (i, 0))],
      out_shape=[jax.ShapeDtypeStruct((N_NODES, F), jnp.bfloat16),
                 jax.ShapeDtypeStruct((N_NODES, 2 * F), jnp.bfloat16)],
  )(x, w_edge.reshape(grid, 1, blk), w_ident.reshape(grid, 1, blk))


def _tc_body(s_ref, r_ref, wt_ref, wl_ref, wr_ref, b_ref,
             t_ref, h_ref, o_ref):
  t = t_ref[0, 0, :][:, None]
  h = h_ref[0, 0, :][:, None]
  s = s_ref[...].astype(jnp.float32)
  r = r_ref[...].astype(jnp.float32)
  dn = (((1,), (1,)), ((), ()))
  acc = lax.dot_general(t * s, wt_ref[...], dn,
                        precision=lax.Precision.HIGHEST,
                        preferred_element_type=jnp.float32)
  acc += lax.dot_general((1.0 - t) * (s - r), wl_ref[...], dn,
                         precision=lax.Precision.HIGHEST,
                         preferred_element_type=jnp.float32)
  acc += lax.dot_general(r, wr_ref[...], dn, precision=lax.Precision.HIGHEST,
                         preferred_element_type=jnp.float32)
  o_ref[...] = jnp.tanh(acc + h * b_ref[0, :][None, :])


def _tc_combine(s, r, wt, wl, wr, bias, twc0, hov):
  blk = 800
  grid = N_NODES // blk
  row_spec = pl.BlockSpec((blk, F), lambda i: (i, 0))
  w_spec = pl.BlockSpec((F, F), lambda i: (0, 0))
  vec_spec = pl.BlockSpec((1, 1, blk), lambda i: (i, 0, 0))
  return pl.pallas_call(
      _tc_body,
      grid=(grid,),
      in_specs=[row_spec, row_spec, w_spec, w_spec, w_spec,
                pl.BlockSpec((1, F), lambda i: (0, 0)),
                vec_spec, vec_spec],
      out_specs=row_spec,
      out_shape=jax.ShapeDtypeStruct((N_NODES, F), jnp.float32),
  )(s, r, wt, wl, wr, bias.reshape(1, F),
    twc0.reshape(grid, 1, blk), hov.reshape(grid, 1, blk))


def kernel(tree_data, W_top, W_left, W_right, bias, pair_center, pair_node,
           twc, lwc, rwc, hov_count):
  n = N_NODES
  parent = jnp.concatenate(
      [jnp.full((1,), n, jnp.int32), pair_center[n:]]).reshape(
          NUM_TILES, NBATCH, SCAT)
  w_edge = jnp.concatenate([jnp.zeros((1,), jnp.float32), rwc[n:]])

  xbf, xcat = _tc_scale(tree_data, w_edge, rwc[:n])
  s_sum, r_sum = _sc_segment_sums(xbf, xcat, parent)
  return _tc_combine(s_sum, r_sum, W_top, W_left, W_right, bias,
                     twc[:n], hov_count)


# speedup vs baseline: 1.1530x; 1.1530x over previous
import functools

import jax
import jax.numpy as jnp
from jax import lax
from jax.experimental import pallas as pl
from jax.experimental.pallas import tpu as pltpu
from jax.experimental.pallas import tpu_sc as plsc

N_NODES = 100000
F = 128
NUM_TILES = 16
EPT = N_NODES // NUM_TILES
SCAT = 125
NBATCH = EPT // SCAT
CH = 625
NCHUNK = EPT // CH
BPC = CH // SCAT
SW = 32
NSLICE = F // SW
ACC_ROWS = N_NODES + 8


def _sc_segment_sums(xbf, xcat, parent3d):
  mesh = plsc.VectorSubcoreMesh(core_axis_name="c", subcore_axis_name="s")

  @functools.partial(
      pl.kernel,
      out_type=[
          jax.ShapeDtypeStruct((N_NODES, F), jnp.bfloat16),
          jax.ShapeDtypeStruct((N_NODES, F), jnp.bfloat16),
      ],
      mesh=mesh,
      compiler_params=pltpu.CompilerParams(use_tc_tiling_on_sc=False,
                                           needs_layout_passes=False),
      scratch_types=[
          pltpu.VMEM_SHARED((ACC_ROWS, SW), jnp.bfloat16),
          pltpu.VMEM((CH, SW), jnp.bfloat16),
          pltpu.VMEM((CH, SW), jnp.bfloat16),
          pltpu.VMEM((NBATCH, SCAT), jnp.int32),
          pltpu.SemaphoreType.DMA,
          pltpu.SemaphoreType.DMA,
          pltpu.SemaphoreType.DMA,
      ],
  )
  def k(x_hbm, xcat_hbm, par_hbm, s_out, r_out, acc, buf0, buf1, idxbuf,
        sem_a, sem_b, sem_s):
    c = lax.axis_index("c")
    s = lax.axis_index("s")
    lo = s * EPT

    pltpu.sync_copy(par_hbm.at[s], idxbuf)

    def init_my_slab(fs):
      @pl.when(c == 0)
      def _():
        pltpu.sync_copy(x_hbm.at[pl.ds(lo, EPT), pl.ds(fs * SW, SW)],
                        acc.at[pl.ds(lo, EPT)])

      @pl.when(c == 1)
      def _():
        pltpu.sync_copy(xcat_hbm.at[pl.ds(lo, EPT), pl.ds(F + fs * SW, SW)],
                        acc.at[pl.ds(lo, EPT)])

    init_my_slab(0)

    for fs in range(NSLICE):
      plsc.subcore_barrier()

      src = [x_hbm, xcat_hbm]
      col = pl.ds(fs * SW, SW)

      @pl.loop(0, NCHUNK // 2)
      def _(p):
        a = 2 * p
        b = 2 * p + 1
        for cc in range(2):
          @pl.when(c == cc)
          def _():
            pltpu.async_copy(
                src[cc].at[pl.ds(lo + a * CH, CH), col], buf0, sem_a)
            pltpu.async_copy(
                src[cc].at[pl.ds(lo + b * CH, CH), col], buf1, sem_b)

        pltpu.make_async_copy(
            x_hbm.at[pl.ds(lo + a * CH, CH), col], buf0, sem_a).wait()
        sc = [pltpu.async_copy(buf0.at[pl.ds(j * SCAT, SCAT)],
                               acc.at[idxbuf.at[a * BPC + j]], sem_s,
                               add=True)
              for j in range(BPC)]
        pltpu.make_async_copy(
            x_hbm.at[pl.ds(lo + b * CH, CH), col], buf1, sem_b).wait()
        sc += [pltpu.async_copy(buf1.at[pl.ds(j * SCAT, SCAT)],
                                acc.at[idxbuf.at[b * BPC + j]], sem_s,
                                add=True)
               for j in range(BPC)]
        for d in sc:
          d.wait()

      plsc.subcore_barrier()

      @pl.when(c == 0)
      def _():
        pltpu.sync_copy(acc.at[pl.ds(lo, EPT)],
                        s_out.at[pl.ds(lo, EPT), pl.ds(fs * SW, SW)])

      @pl.when(c == 1)
      def _():
        pltpu.sync_copy(acc.at[pl.ds(lo, EPT)],
                        r_out.at[pl.ds(lo, EPT), pl.ds(fs * SW, SW)])

      if fs != NSLICE - 1:
        init_my_slab(fs + 1)

  return k(xbf, xcat, parent3d)


def _scale_body(x_ref, we_ref, w0_ref, xb_ref, o_ref):
  xb = x_ref[...]
  xb_ref[...] = xb.astype(jnp.bfloat16)
  o_ref[:, :F] = (xb * we_ref[0, 0, :][:, None]).astype(jnp.bfloat16)
  o_ref[:, F:] = (xb * w0_ref[0, 0, :][:, None]).astype(jnp.bfloat16)


def _tc_scale(x, w_edge, w_ident):
  blk = 2000
  grid = N_NODES // blk
  vec_spec = pl.BlockSpec((1, 1, blk), lambda i: (i, 0, 0))
  return pl.pallas_call(
      _scale_body,
      grid=(grid,),
      in_specs=[pl.BlockSpec((blk, F), lambda i: (i, 0)), vec_spec, vec_spec],
      out_specs=[pl.BlockSpec((blk, F), lambda i: (i, 0)),
                 pl.BlockSpec((blk, 2 * F), lambda i: (i, 0))],
      out_shape=[jax.ShapeDtypeStruct((N_NODES, F), jnp.bfloat16),
                 jax.ShapeDtypeStruct((N_NODES, 2 * F), jnp.bfloat16)],
  )(x, w_edge.reshape(grid, 1, blk), w_ident.reshape(grid, 1, blk))


def _tc_body(s_ref, r_ref, w_ref, b_ref, t_ref, h_ref, o_ref):
  t = t_ref[0, 0, :][:, None]
  h = h_ref[0, 0, :][:, None]
  s = s_ref[...]
  r = r_ref[...]
  dn = (((1,), (1,)), ((), ()))

  def proj(v, k):
    hi = lax.dot_general(v, w_ref[2 * k], dn,
                         preferred_element_type=jnp.float32)
    lo = lax.dot_general(v, w_ref[2 * k + 1], dn,
                         preferred_element_type=jnp.float32)
    return hi + lo

  a = proj(s, 0)
  bs = proj(s, 1)
  br = proj(r, 1)
  cr = proj(r, 2)
  acc = t * (a - bs + br) + bs - br + cr
  o_ref[...] = jnp.tanh(acc + h * b_ref[0, :][None, :])


def _tc_combine(s, r, wsplit, bias, twc0, hov):
  blk = 4000
  grid = N_NODES // blk
  row_spec = pl.BlockSpec((blk, F), lambda i: (i, 0))
  vec_spec = pl.BlockSpec((1, 1, blk), lambda i: (i, 0, 0))
  return pl.pallas_call(
      _tc_body,
      grid=(grid,),
      in_specs=[row_spec, row_spec,
                pl.BlockSpec((6, F, F), lambda i: (0, 0, 0)),
                pl.BlockSpec((1, F), lambda i: (0, 0)),
                vec_spec, vec_spec],
      out_specs=row_spec,
      out_shape=jax.ShapeDtypeStruct((N_NODES, F), jnp.float32),
  )(s, r, wsplit, bias.reshape(1, F),
    twc0.reshape(grid, 1, blk), hov.reshape(grid, 1, blk))


def kernel(tree_data, W_top, W_left, W_right, bias, pair_center, pair_node,
           twc, lwc, rwc, hov_count):
  n = N_NODES
  parent = jnp.concatenate(
      [jnp.full((1,), n, jnp.int32), pair_center[n:]]).reshape(
          NUM_TILES, NBATCH, SCAT)
  w_edge = jnp.concatenate([jnp.zeros((1,), jnp.float32), rwc[n:]])

  ws = jnp.stack([W_top, W_left, W_right])
  hi = ws.astype(jnp.bfloat16)
  lo = (ws - hi.astype(jnp.float32)).astype(jnp.bfloat16)
  wsplit = jnp.stack([hi[0], lo[0], hi[1], lo[1], hi[2], lo[2]])

  xbf, xcat = _tc_scale(tree_data, w_edge, rwc[:n])
  s_sum, r_sum = _sc_segment_sums(xbf, xcat, parent)
  return _tc_combine(s_sum, r_sum, wsplit, bias, twc[:n], hov_count)


# speedup vs baseline: 1.2886x; 1.1177x over previous
import functools

import jax
import jax.numpy as jnp
from jax import lax
from jax.experimental import pallas as pl
from jax.experimental.pallas import tpu as pltpu
from jax.experimental.pallas import tpu_sc as plsc

N_NODES = 100000
F = 128
NUM_TILES = 16
EPT = N_NODES // NUM_TILES
SCAT = 125
NBATCH = EPT // SCAT
CH = 625
NCHUNK = EPT // CH
BPC = CH // SCAT
NSLICE = F // 16
ACC_ROWS = N_NODES + 8


def _sc_segment_sums(x, xcat, parent3d):
  mesh = plsc.VectorSubcoreMesh(core_axis_name="c", subcore_axis_name="s")

  @functools.partial(
      pl.kernel,
      out_type=[
          jax.ShapeDtypeStruct((N_NODES, F), jnp.float32),
          jax.ShapeDtypeStruct((N_NODES, F), jnp.float32),
      ],
      mesh=mesh,
      compiler_params=pltpu.CompilerParams(use_tc_tiling_on_sc=False,
                                           needs_layout_passes=False),
      scratch_types=[
          pltpu.VMEM_SHARED((ACC_ROWS, 16), jnp.float32),
          pltpu.VMEM((CH, 16), jnp.float32),
          pltpu.VMEM((CH, 16), jnp.float32),
          pltpu.VMEM((NBATCH, SCAT), jnp.int32),
          pltpu.SemaphoreType.DMA,
          pltpu.SemaphoreType.DMA,
          pltpu.SemaphoreType.DMA,
      ],
  )
  def k(x_hbm, xcat_hbm, par_hbm, s_out, r_out, acc, buf0, buf1, idxbuf,
        sem_a, sem_b, sem_s):
    c = lax.axis_index("c")
    s = lax.axis_index("s")
    lo = s * EPT

    pltpu.sync_copy(par_hbm.at[s], idxbuf)

    def init_my_slab(fs):
      @pl.when(c == 0)
      def _():
        pltpu.sync_copy(x_hbm.at[pl.ds(lo, EPT), pl.ds(fs * 16, 16)],
                        acc.at[pl.ds(lo, EPT)])

      @pl.when(c == 1)
      def _():
        pltpu.sync_copy(xcat_hbm.at[pl.ds(lo, EPT), pl.ds(F + fs * 16, 16)],
                        acc.at[pl.ds(lo, EPT)])

    init_my_slab(0)

    for fs in range(NSLICE):
      plsc.subcore_barrier()

      src = [x_hbm, xcat_hbm]
      col = pl.ds(fs * 16, 16)

      @pl.loop(0, NCHUNK // 2)
      def _(p):
        a = 2 * p
        b = 2 * p + 1
        for cc in range(2):
          @pl.when(c == cc)
          def _():
            pltpu.async_copy(
                src[cc].at[pl.ds(lo + a * CH, CH), col], buf0, sem_a)
            pltpu.async_copy(
                src[cc].at[pl.ds(lo + b * CH, CH), col], buf1, sem_b)

        pltpu.make_async_copy(
            x_hbm.at[pl.ds(lo + a * CH, CH), col], buf0, sem_a).wait()
        sc = [pltpu.async_copy(buf0.at[pl.ds(j * SCAT, SCAT)],
                               acc.at[idxbuf.at[a * BPC + j]], sem_s,
                               add=True)
              for j in range(BPC)]
        pltpu.make_async_copy(
            x_hbm.at[pl.ds(lo + b * CH, CH), col], buf1, sem_b).wait()
        sc += [pltpu.async_copy(buf1.at[pl.ds(j * SCAT, SCAT)],
                                acc.at[idxbuf.at[b * BPC + j]], sem_s,
                                add=True)
               for j in range(BPC)]
        for d in sc:
          d.wait()

      plsc.subcore_barrier()

      @pl.when(c == 0)
      def _():
        pltpu.sync_copy(acc.at[pl.ds(lo, EPT)],
                        s_out.at[pl.ds(lo, EPT), pl.ds(fs * 16, 16)])

      @pl.when(c == 1)
      def _():
        pltpu.sync_copy(acc.at[pl.ds(lo, EPT)],
                        r_out.at[pl.ds(lo, EPT), pl.ds(fs * 16, 16)])

      if fs != NSLICE - 1:
        init_my_slab(fs + 1)

  return k(x, xcat, parent3d)


def _scale_body(x_ref, we_ref, w0_ref, o_ref):
  xb = x_ref[...]
  o_ref[:, :F] = xb * we_ref[0, 0, :][:, None]
  o_ref[:, F:] = xb * w0_ref[0, 0, :][:, None]


def _tc_scale(x, w_edge, w_ident):
  blk = 2000
  grid = N_NODES // blk
  vec_spec = pl.BlockSpec((1, 1, blk), lambda i: (i, 0, 0))
  return pl.pallas_call(
      _scale_body,
      grid=(grid,),
      in_specs=[pl.BlockSpec((blk, F), lambda i: (i, 0)), vec_spec, vec_spec],
      out_specs=pl.BlockSpec((blk, 2 * F), lambda i: (i, 0)),
      out_shape=jax.ShapeDtypeStruct((N_NODES, 2 * F), jnp.float32),
  )(x, w_edge.reshape(grid, 1, blk), w_ident.reshape(grid, 1, blk))


def _tc_body(s_ref, r_ref, wt_ref, wl_ref, wr_ref, b_ref,
             t_ref, h_ref, o_ref):
  t = t_ref[0, 0, :][:, None]
  h = h_ref[0, 0, :][:, None]
  s = s_ref[...]
  r = r_ref[...]
  dn = (((1,), (1,)), ((), ()))
  acc = lax.dot_general(t * s, wt_ref[...], dn, precision=lax.Precision.DEFAULT,
                        preferred_element_type=jnp.float32)
  acc += lax.dot_general((1.0 - t) * (s - r), wl_ref[...], dn,
                         precision=lax.Precision.DEFAULT,
                         preferred_element_type=jnp.float32)
  acc += lax.dot_general(r, wr_ref[...], dn, precision=lax.Precision.DEFAULT,
                         preferred_element_type=jnp.float32)
  o_ref[...] = jnp.tanh(acc + h * b_ref[0, :][None, :])


def _tc_combine(s, r, wt, wl, wr, bias, twc0, hov):
  blk = 4000
  grid = N_NODES // blk
  row_spec = pl.BlockSpec((blk, F), lambda i: (i, 0))
  w_spec = pl.BlockSpec((F, F), lambda i: (0, 0))
  vec_spec = pl.BlockSpec((1, 1, blk), lambda i: (i, 0, 0))
  return pl.pallas_call(
      _tc_body,
      grid=(grid,),
      in_specs=[row_spec, row_spec, w_spec, w_spec, w_spec,
                pl.BlockSpec((1, F), lambda i: (0, 0)),
                vec_spec, vec_spec],
      out_specs=row_spec,
      out_shape=jax.ShapeDtypeStruct((N_NODES, F), jnp.float32),
  )(s, r, wt, wl, wr, bias.reshape(1, F),
    twc0.reshape(grid, 1, blk), hov.reshape(grid, 1, blk))


def kernel(tree_data, W_top, W_left, W_right, bias, pair_center, pair_node,
           twc, lwc, rwc, hov_count):
  n = N_NODES
  parent = jnp.concatenate(
      [jnp.full((1,), n, jnp.int32), pair_center[n:]]).reshape(
          NUM_TILES, NBATCH, SCAT)
  w_edge = jnp.concatenate([jnp.zeros((1,), jnp.float32), rwc[n:]])

  xcat = _tc_scale(tree_data, w_edge, rwc[:n])
  s_sum, r_sum = _sc_segment_sums(tree_data, xcat, parent)
  return _tc_combine(s_sum, r_sum, W_top, W_left, W_right, bias,
                     twc[:n], hov_count)


# speedup vs baseline: 1.3014x; 1.0099x over previous
import functools

import jax
import jax.numpy as jnp
from jax import lax
from jax.experimental import pallas as pl
from jax.experimental.pallas import tpu as pltpu
from jax.experimental.pallas import tpu_sc as plsc

N_NODES = 100000
F = 128
NUM_TILES = 16
EPT = N_NODES // NUM_TILES
SCAT = 125
NBATCH = EPT // SCAT
CH = 625
NCHUNK = EPT // CH
BPC = CH // SCAT
NSLICE = F // 16
ACC_ROWS = N_NODES + 8


def _sc_segment_sums(x, xcat, parent3d):
  mesh = plsc.VectorSubcoreMesh(core_axis_name="c", subcore_axis_name="s")

  @functools.partial(
      pl.kernel,
      out_type=[
          jax.ShapeDtypeStruct((N_NODES, F), jnp.float32),
          jax.ShapeDtypeStruct((N_NODES, F), jnp.float32),
      ],
      mesh=mesh,
      compiler_params=pltpu.CompilerParams(use_tc_tiling_on_sc=False,
                                           needs_layout_passes=False),
      scratch_types=[
          pltpu.VMEM_SHARED((ACC_ROWS, 16), jnp.float32),
          pltpu.VMEM((CH, 16), jnp.float32),
          pltpu.VMEM((CH, 16), jnp.float32),
          pltpu.VMEM((NBATCH, SCAT), jnp.int32),
          pltpu.SemaphoreType.DMA,
          pltpu.SemaphoreType.DMA,
          pltpu.SemaphoreType.DMA,
      ],
  )
  def k(x_hbm, xcat_hbm, par_hbm, s_out, r_out, acc, buf0, buf1, idxbuf,
        sem_a, sem_b, sem_s):
    c = lax.axis_index("c")
    s = lax.axis_index("s")
    lo = s * EPT

    pltpu.sync_copy(par_hbm.at[s], idxbuf)

    def init_my_slab(fs):
      @pl.when(c == 0)
      def _():
        pltpu.sync_copy(x_hbm.at[pl.ds(lo, EPT), pl.ds(fs * 16, 16)],
                        acc.at[pl.ds(lo, EPT)])

      @pl.when(c == 1)
      def _():
        pltpu.sync_copy(xcat_hbm.at[pl.ds(lo, EPT), pl.ds(F + fs * 16, 16)],
                        acc.at[pl.ds(lo, EPT)])

    init_my_slab(0)

    for fs in range(NSLICE):
      plsc.subcore_barrier()

      src = [x_hbm, xcat_hbm]
      col = pl.ds(fs * 16, 16)

      @pl.loop(0, NCHUNK // 2)
      def _(p):
        a = 2 * p
        b = 2 * p + 1
        for cc in range(2):
          @pl.when(c == cc)
          def _():
            pltpu.async_copy(
                src[cc].at[pl.ds(lo + a * CH, CH), col], buf0, sem_a)
            pltpu.async_copy(
                src[cc].at[pl.ds(lo + b * CH, CH), col], buf1, sem_b)

        pltpu.make_async_copy(
            x_hbm.at[pl.ds(lo + a * CH, CH), col], buf0, sem_a).wait()
        sc = [pltpu.async_copy(buf0.at[pl.ds(j * SCAT, SCAT)],
                               acc.at[idxbuf.at[a * BPC + j]], sem_s,
                               add=True)
              for j in range(BPC)]
        pltpu.make_async_copy(
            x_hbm.at[pl.ds(lo + b * CH, CH), col], buf1, sem_b).wait()
        sc += [pltpu.async_copy(buf1.at[pl.ds(j * SCAT, SCAT)],
                                acc.at[idxbuf.at[b * BPC + j]], sem_s,
                                add=True)
               for j in range(BPC)]
        for d in sc:
          d.wait()

      plsc.subcore_barrier()

      @pl.when(c == 0)
      def _():
        pltpu.sync_copy(acc.at[pl.ds(lo, EPT)],
                        s_out.at[pl.ds(lo, EPT), pl.ds(fs * 16, 16)])

      @pl.when(c == 1)
      def _():
        pltpu.sync_copy(acc.at[pl.ds(lo, EPT)],
                        r_out.at[pl.ds(lo, EPT), pl.ds(fs * 16, 16)])

      if fs != NSLICE - 1:
        init_my_slab(fs + 1)

  return k(x, xcat, parent3d)


def _scale_body(x_ref, we_ref, w0_ref, o_ref):
  xb = x_ref[...]
  o_ref[:, :F] = xb * we_ref[0, 0, :][:, None]
  o_ref[:, F:] = xb * w0_ref[0, 0, :][:, None]


def _tc_scale(x, w_edge, w_ident):
  blk = 4000
  grid = N_NODES // blk
  vec_spec = pl.BlockSpec((1, 1, blk), lambda i: (i, 0, 0))
  return pl.pallas_call(
      _scale_body,
      grid=(grid,),
      in_specs=[pl.BlockSpec((blk, F), lambda i: (i, 0)), vec_spec, vec_spec],
      out_specs=pl.BlockSpec((blk, 2 * F), lambda i: (i, 0)),
      out_shape=jax.ShapeDtypeStruct((N_NODES, 2 * F), jnp.float32),
  )(x, w_edge.reshape(grid, 1, blk), w_ident.reshape(grid, 1, blk))


def _tc_body(s_ref, r_ref, wt_ref, wl_ref, wr_ref, b_ref,
             t_ref, h_ref, o_ref):
  t = t_ref[0, 0, :][:, None]
  h = h_ref[0, 0, :][:, None]
  s = s_ref[...]
  r = r_ref[...]
  dn = (((1,), (1,)), ((), ()))
  acc = lax.dot_general(t * s, wt_ref[...], dn, precision=lax.Precision.DEFAULT,
                        preferred_element_type=jnp.float32)
  acc += lax.dot_general((1.0 - t) * (s - r), wl_ref[...], dn,
                         precision=lax.Precision.DEFAULT,
                         preferred_element_type=jnp.float32)
  acc += lax.dot_general(r, wr_ref[...], dn, precision=lax.Precision.DEFAULT,
                         preferred_element_type=jnp.float32)
  o_ref[...] = jnp.tanh(acc + h * b_ref[0, :][None, :])


def _tc_combine(s, r, wt, wl, wr, bias, twc0, hov):
  blk = 5000
  grid = N_NODES // blk
  row_spec = pl.BlockSpec((blk, F), lambda i: (i, 0))
  w_spec = pl.BlockSpec((F, F), lambda i: (0, 0))
  vec_spec = pl.BlockSpec((1, 1, blk), lambda i: (i, 0, 0))
  return pl.pallas_call(
      _tc_body,
      grid=(grid,),
      in_specs=[row_spec, row_spec, w_spec, w_spec, w_spec,
                pl.BlockSpec((1, F), lambda i: (0, 0)),
                vec_spec, vec_spec],
      out_specs=row_spec,
      out_shape=jax.ShapeDtypeStruct((N_NODES, F), jnp.float32),
  )(s, r, wt, wl, wr, bias.reshape(1, F),
    twc0.reshape(grid, 1, blk), hov.reshape(grid, 1, blk))


def kernel(tree_data, W_top, W_left, W_right, bias, pair_center, pair_node,
           twc, lwc, rwc, hov_count):
  n = N_NODES
  parent = jnp.concatenate(
      [jnp.full((1,), n, jnp.int32), pair_center[n:]]).reshape(
          NUM_TILES, NBATCH, SCAT)
  w_edge = jnp.concatenate([jnp.zeros((1,), jnp.float32), rwc[n:]])

  xcat = _tc_scale(tree_data, w_edge, rwc[:n])
  s_sum, r_sum = _sc_segment_sums(tree_data, xcat, parent)
  return _tc_combine(s_sum, r_sum, W_top, W_left, W_right, bias,
                     twc[:n], hov_count)
